# baseline (speedup 1.0000x reference)
import functools

import jax
import jax.numpy as jnp
from jax import lax
from jax.experimental import pallas as pl
from jax.experimental.pallas import tpu as pltpu
from jax.experimental.pallas import tpu_sc as plsc

_NC = 2
_NS = 16
_NW = _NC * _NS
_L = 16


def _build_plane_gather(F, V, D, B):
    n_planes = F * D
    ppw = n_planes // _NW
    qb = B // 4

    mesh = plsc.VectorSubcoreMesh(core_axis_name="c", subcore_axis_name="s")

    @functools.partial(
        pl.kernel,
        mesh=mesh,
        compiler_params=pltpu.CompilerParams(needs_layout_passes=False),
        out_type=jax.ShapeDtypeStruct((n_planes, B), jnp.float32),
        scratch_types=[
            pltpu.VMEM((V,), jnp.float32),
            pltpu.VMEM((B,), jnp.int32),
            pltpu.VMEM((qb,), jnp.float32),
            pltpu.VMEM((qb,), jnp.float32),
            pltpu.SemaphoreType.DMA,
            pltpu.SemaphoreType.DMA,
        ],
    )
    def plane_kernel(tab_hbm, idx_hbm, out_hbm, plane_v, idx_v,
                     row_a, row_b, osem_a, osem_b):
        wid = lax.axis_index("s") * _NC + lax.axis_index("c")
        r0 = wid * ppw
        rows = (row_a, row_b)
        osems = (osem_a, osem_b)

        def body(i, f_prev):
            r = r0 + i
            f = r // D

            @pl.when(f != f_prev)
            def _():
                pltpu.sync_copy(idx_hbm.at[pl.ds(f * B, B)], idx_v)

            pltpu.sync_copy(tab_hbm.at[f, r % D, :], plane_v)

            writes = [None, None, None, None]
            for q in range(4):
                row_v = rows[q % 2]
                if q >= 2:
                    writes[q - 2].wait()

                def gloop(j, c2, _q=q, _row=row_v):
                    sl = pl.ds(j * _L, _L)
                    _row[sl] = plsc.load_gather(
                        plane_v, [idx_v[pl.ds(_q * qb + j * _L, _L)]])
                    return c2

                lax.fori_loop(0, qb // _L, gloop, 0, unroll=8)
                writes[q] = pltpu.async_copy(
                    row_v, out_hbm.at[r, pl.ds(q * qb, qb)], osems[q % 2])
            writes[2].wait()
            writes[3].wait()
            return f

        lax.fori_loop(0, ppw, body, jnp.int32(-1), unroll=False)

    return plane_kernel


def kernel(inputs, tables):
    F, V, D = tables.shape
    B = inputs.shape[0]
    tab_t = jnp.transpose(tables, (0, 2, 1))
    idx_f = jnp.transpose(inputs, (1, 0)).reshape(F * B)
    out = _build_plane_gather(F, V, D, B)(tab_t, idx_f.astype(jnp.int32))
    return out.reshape(F, D, B).transpose(2, 0, 1)


# speedup vs baseline: 2.0259x; 2.0259x over previous
import functools

import jax
import jax.numpy as jnp
from jax import lax
from jax.experimental import pallas as pl
from jax.experimental.pallas import tpu as pltpu
from jax.experimental.pallas import tpu_sc as plsc

_NC = 2
_NS = 16
_NW = _NC * _NS
_L = 16


def _build_plane_gather(F, V, D, B):
    n_planes = F * D
    ppw = n_planes // _NW
    qb = B // 4

    mesh = plsc.VectorSubcoreMesh(core_axis_name="c", subcore_axis_name="s")

    @functools.partial(
        pl.kernel,
        mesh=mesh,
        compiler_params=pltpu.CompilerParams(needs_layout_passes=False),
        out_type=jax.ShapeDtypeStruct((n_planes, B), jnp.float32),
        scratch_types=[
            pltpu.VMEM((V,), jnp.float32),
            pltpu.VMEM((B,), jnp.int32),
            pltpu.VMEM((qb,), jnp.float32),
            pltpu.VMEM((qb,), jnp.float32),
            pltpu.SemaphoreType.DMA,
            pltpu.SemaphoreType.DMA,
        ],
    )
    def plane_kernel(tab_hbm, idx_hbm, out_hbm, plane_v, idx_v,
                     row_a, row_b, osem_a, osem_b):
        wid = lax.axis_index("s") * _NC + lax.axis_index("c")
        r0 = wid * ppw
        rows = (row_a, row_b)
        osems = (osem_a, osem_b)

        def body(i, f_prev):
            r = r0 + i
            f = r // D

            @pl.when(f != f_prev)
            def _():
                pltpu.sync_copy(idx_hbm.at[pl.ds(f * B, B)], idx_v)

            pltpu.sync_copy(tab_hbm.at[f, r % D, :], plane_v)

            writes = [None, None, None, None]
            for q in range(4):
                row_v = rows[q % 2]
                if q >= 2:
                    writes[q - 2].wait()

                U = 8

                def gloop(j, c2, _q=q, _row=row_v):
                    base = j * _L * U
                    ivs = [idx_v[pl.ds(_q * qb + base + k * _L, _L)]
                           for k in range(U)]
                    gs = [plsc.load_gather(plane_v, [iv]) for iv in ivs]
                    for k in range(U):
                        _row[pl.ds(base + k * _L, _L)] = gs[k]
                    return c2

                lax.fori_loop(0, qb // (_L * U), gloop, 0, unroll=False)
                writes[q] = pltpu.async_copy(
                    row_v, out_hbm.at[r, pl.ds(q * qb, qb)], osems[q % 2])
            writes[2].wait()
            writes[3].wait()
            return f

        lax.fori_loop(0, ppw, body, jnp.int32(-1), unroll=False)

    return plane_kernel


def kernel(inputs, tables):
    F, V, D = tables.shape
    B = inputs.shape[0]
    tab_t = jnp.transpose(tables, (0, 2, 1))
    idx_f = jnp.transpose(inputs, (1, 0)).reshape(F * B)
    out = _build_plane_gather(F, V, D, B)(tab_t, idx_f.astype(jnp.int32))
    return out.reshape(F, D, B).transpose(2, 0, 1)


# speedup vs baseline: 2.0346x; 1.0043x over previous
import functools

import jax
import jax.numpy as jnp
from jax import lax
from jax.experimental import pallas as pl
from jax.experimental.pallas import tpu as pltpu
from jax.experimental.pallas import tpu_sc as plsc

_NC = 2
_NS = 16
_NW = _NC * _NS
_L = 16


def _build_plane_gather(F, V, D, B):
    n_planes = F * D
    ppw = n_planes // _NW
    qb = B // 4

    mesh = plsc.VectorSubcoreMesh(core_axis_name="c", subcore_axis_name="s")

    @functools.partial(
        pl.kernel,
        mesh=mesh,
        compiler_params=pltpu.CompilerParams(needs_layout_passes=False),
        out_type=jax.ShapeDtypeStruct((n_planes, B), jnp.float32),
        scratch_types=[
            pltpu.VMEM((V,), jnp.float32),
            pltpu.VMEM((B,), jnp.int32),
            pltpu.VMEM((qb,), jnp.float32),
            pltpu.VMEM((qb,), jnp.float32),
            pltpu.SemaphoreType.DMA,
            pltpu.SemaphoreType.DMA,
        ],
    )
    def plane_kernel(tab_hbm, idx_hbm, out_hbm, plane_v, idx_v,
                     row_a, row_b, osem_a, osem_b):
        wid = lax.axis_index("s") * _NC + lax.axis_index("c")
        r0 = wid * ppw
        rows = (row_a, row_b)
        osems = (osem_a, osem_b)

        def body(i, f_prev):
            r = r0 + i
            f = r // D

            @pl.when(f != f_prev)
            def _():
                pltpu.sync_copy(idx_hbm.at[pl.ds(f * B, B)], idx_v)

            pltpu.sync_copy(tab_hbm.at[f, r % D, :], plane_v)

            writes = [None, None, None, None]
            for q in range(4):
                row_v = rows[q % 2]
                if q >= 2:
                    writes[q - 2].wait()

                U = 16

                def gloop(j, c2, _q=q, _row=row_v):
                    base = j * _L * U
                    ivs = [idx_v[pl.ds(_q * qb + base + k * _L, _L)]
                           for k in range(U)]
                    gs = [plsc.load_gather(plane_v, [iv]) for iv in ivs]
                    for k in range(U):
                        _row[pl.ds(base + k * _L, _L)] = gs[k]
                    return c2

                lax.fori_loop(0, qb // (_L * U), gloop, 0, unroll=False)
                writes[q] = pltpu.async_copy(
                    row_v, out_hbm.at[r, pl.ds(q * qb, qb)], osems[q % 2])
            writes[2].wait()
            writes[3].wait()
            return f

        lax.fori_loop(0, ppw, body, jnp.int32(-1), unroll=False)

    return plane_kernel


def kernel(inputs, tables):
    F, V, D = tables.shape
    B = inputs.shape[0]
    tab_t = jnp.transpose(tables, (0, 2, 1))
    idx_f = jnp.transpose(inputs, (1, 0)).reshape(F * B)
    out = _build_plane_gather(F, V, D, B)(tab_t, idx_f.astype(jnp.int32))
    return out.reshape(F, D, B).transpose(2, 0, 1)


# speedup vs baseline: 2.0810x; 1.0228x over previous
import functools

import jax
import jax.numpy as jnp
from jax import lax
from jax.experimental import pallas as pl
from jax.experimental.pallas import tpu as pltpu
from jax.experimental.pallas import tpu_sc as plsc

_NC = 2
_NS = 16
_NW = _NC * _NS
_L = 16


def _build_plane_gather(F, V, D, B):
    n_planes = F * D
    ppw = n_planes // _NW
    qb = B // 4

    mesh = plsc.VectorSubcoreMesh(core_axis_name="c", subcore_axis_name="s")

    @functools.partial(
        pl.kernel,
        mesh=mesh,
        compiler_params=pltpu.CompilerParams(needs_layout_passes=False),
        out_type=jax.ShapeDtypeStruct((n_planes, B), jnp.float32),
        scratch_types=[
            pltpu.VMEM((V,), jnp.float32),
            pltpu.VMEM((B,), jnp.int32),
            pltpu.VMEM((qb,), jnp.float32),
            pltpu.VMEM((qb,), jnp.float32),
            pltpu.SemaphoreType.DMA,
            pltpu.SemaphoreType.DMA,
            pltpu.SemaphoreType.DMA,
        ],
    )
    def plane_kernel(tab_hbm, idx_hbm, out_hbm, plane_v, idx_v,
                     row_a, row_b, osem_a, osem_b, psem):
        wid = lax.axis_index("s") * _NC + lax.axis_index("c")
        r0 = wid * ppw
        rows = (row_a, row_b)
        osems = (osem_a, osem_b)

        pltpu.async_copy(tab_hbm.at[r0 // D, r0 % D, :], plane_v, psem)

        def body(i, f_prev):
            r = r0 + i
            f = r // D

            @pl.when(f != f_prev)
            def _():
                pltpu.sync_copy(idx_hbm.at[pl.ds(f * B, B)], idx_v)

            pltpu.make_async_copy(tab_hbm.at[f, r % D, :], plane_v,
                                  psem).wait()

            writes = [None, None, None, None]
            for q in range(4):
                row_v = rows[q % 2]
                if q >= 2:
                    writes[q - 2].wait()

                U = 16

                def gloop(j, c2, _q=q, _row=row_v):
                    base = j * _L * U
                    ivs = [idx_v[pl.ds(_q * qb + base + k * _L, _L)]
                           for k in range(U)]
                    gs = [plsc.load_gather(plane_v, [iv]) for iv in ivs]
                    for k in range(U):
                        _row[pl.ds(base + k * _L, _L)] = gs[k]
                    return c2

                lax.fori_loop(0, qb // (_L * U), gloop, 0, unroll=False)
                writes[q] = pltpu.async_copy(
                    row_v, out_hbm.at[r, pl.ds(q * qb, qb)], osems[q % 2])

            @pl.when(i + 1 < ppw)
            def _():
                rn = r + 1
                pltpu.async_copy(tab_hbm.at[rn // D, rn % D, :], plane_v,
                                 psem)

            writes[2].wait()
            writes[3].wait()
            return f

        lax.fori_loop(0, ppw, body, jnp.int32(-1), unroll=False)

    return plane_kernel


def kernel(inputs, tables):
    F, V, D = tables.shape
    B = inputs.shape[0]
    tab_t = jnp.transpose(tables, (0, 2, 1))
    idx_f = jnp.transpose(inputs, (1, 0)).reshape(F * B)
    out = _build_plane_gather(F, V, D, B)(tab_t, idx_f.astype(jnp.int32))
    return out.reshape(F, D, B).transpose(2, 0, 1)
